# baseline (speedup 1.0000x reference)
import functools

import jax
import jax.numpy as jnp
from jax import lax
from jax.experimental import pallas as pl
from jax.experimental.pallas import tpu as pltpu
from jax.experimental.pallas import tpu_sc as plsc

N = 10000
E = 320000
D = 128
G = 8
NC = 2
NS = 16
NW = NC * NS

CH = 128
NCHUNK = E // CH
NPART = 2
CPP = NCHUNK // NPART
EPP = CPP * CH
WB = CPP // NW
WR = CPP % NW
CSCP = CPP // NC
TB = CSCP // NS
TR = CSCP % NS
NPAD = 10240
NROWS_TILE = NPAD // NS
FLUSH_CHUNK = 128

NSLOT = 2
HALF = 1
NGROUP = -(-(WB + 1) // NSLOT)
NSLOT_S = 2
NGROUP_S = -(-(TB + 1) // NSLOT_S)

_MESH = dict(core_axis_name="c", subcore_axis_name="s",
             num_cores=NC, num_subcores=NS)


def _dx_body(x_ref, w_ref, b_ref, o_ref):
    o_ref[...] = (
        jnp.dot(x_ref[...], w_ref[...], preferred_element_type=jnp.float32)
        + b_ref[...]
    )


def _dx_call(x, w1t, b1):
    return pl.pallas_call(
        _dx_body,
        out_shape=jax.ShapeDtypeStruct((N, D), jnp.float32),
    )(x, w1t, b1)


def _gather_body(part, dx_hbm, idx_hbm, out_hbm, idx1, rows, dxs,
                 sgA, sgB, ssA, ssB):
    c = lax.axis_index("c")
    s = lax.axis_index("s")
    wid = s * NC + c
    n = WB + (wid < WR).astype(jnp.int32)
    loc = wid * WB + jnp.minimum(wid, WR)
    base = part * CPP + loc

    @pl.when(s < 10)
    def _():
        pltpu.sync_copy(
            dx_hbm.at[pl.ds(s * (N // 10), N // 10)],
            dxs.at[pl.ds(s * (N // 10), N // 10)],
        )

    pltpu.sync_copy(
        idx_hbm.at[pl.ds(base * CH, WB * CH)], idx1.at[pl.ds(0, WB * CH)]
    )
    plsc.subcore_barrier()

    @pl.when(n > WB)
    def _():
        pltpu.sync_copy(
            idx_hbm.at[pl.ds((base + WB) * CH, CH)],
            idx1.at[pl.ds(WB * CH, CH)],
        )

    def g_desc(slot, j):
        return pltpu.make_async_copy(
            dxs.at[idx1.at[pl.ds(j * CH, CH)]], rows.at[slot],
            sgA if slot < HALF else sgB,
        )

    def s_desc(slot, j):
        return pltpu.make_async_copy(
            rows.at[slot], out_hbm.at[pl.ds((loc + j) * CH, CH)],
            ssA if slot < HALF else ssB,
        )

    def fire_g(slot, j):
        @pl.when(j < n)
        def _():
            g_desc(slot, j).start()

    def drain_g(slot, j):
        @pl.when(j < n)
        def _():
            g_desc(slot, j).wait()

    def fire_s(slot, j):
        @pl.when(j < n)
        def _():
            s_desc(slot, j).start()

    def drain_s(slot, j):
        @pl.when((j >= 0) & (j < n))
        def _():
            s_desc(slot, j).wait()

    def group(gi, carry):
        q = gi * NSLOT
        for b in range(HALF):
            drain_s(b, q - NSLOT + b)
            fire_g(b, q + b)
        for b in range(HALF, NSLOT):
            drain_s(b, q - NSLOT + b)
            fire_g(b, q + b)
        for b in range(HALF):
            drain_g(b, q + b)
            fire_s(b, q + b)
        for b in range(HALF, NSLOT):
            drain_g(b, q + b)
            fire_s(b, q + b)
        return carry

    lax.fori_loop(0, NGROUP, group, 0)
    q = NGROUP * NSLOT
    for b in range(NSLOT):
        drain_s(b, q - NSLOT + b)


@functools.cache
def _gather_kernel(part):
    return pl.kernel(
        functools.partial(_gather_body, part),
        mesh=plsc.VectorSubcoreMesh(**_MESH),
        out_type=jax.ShapeDtypeStruct((EPP, D), jnp.float32),
        scratch_types=[
            pltpu.VMEM(((WB + 2) * CH,), jnp.int32),
            pltpu.VMEM((NSLOT, CH, D), jnp.float32),
            pltpu.VMEM_SHARED((N, D), jnp.float32),
            pltpu.SemaphoreType.DMA,
            pltpu.SemaphoreType.DMA,
            pltpu.SemaphoreType.DMA,
            pltpu.SemaphoreType.DMA,
        ],
        name=f"gine_gather_p{part}",
    )


def _msg_body(g_ref, e_ref, w_ref, b_ref, o_ref):
    v = (
        g_ref[...]
        + jnp.dot(e_ref[...], w_ref[...], preferred_element_type=jnp.float32)
        + b_ref[...]
    )
    o_ref[...] = v * (1.0 / (1.0 + jnp.exp(-v)))


def _msg_call(part, gathered, e, cwt, cb):
    blk = 4000
    grid = EPP // blk
    off = part * grid
    return pl.pallas_call(
        _msg_body,
        grid=(grid,),
        in_specs=[
            pl.BlockSpec((blk, D), lambda i: (i, 0)),
            pl.BlockSpec((blk, D), lambda i, off=off: (i + off, 0)),
            pl.BlockSpec((D, D), lambda i: (0, 0)),
            pl.BlockSpec((1, D), lambda i: (0, 0)),
        ],
        out_specs=pl.BlockSpec((blk, D), lambda i: (i, 0)),
        out_shape=jax.ShapeDtypeStruct((EPP, D), jnp.float32),
    )(gathered, e, cwt, cb)


def _scatter_body(part, msg_hbm, idx_hbm, out_hbm, idx1, rows, agg_sh,
                  sfA, sfB, saA, saB):
    c = lax.axis_index("c")
    s = lax.axis_index("s")

    def zrow(i, carry):
        for k in range(D // 16):
            rows[0, i, pl.ds(k * 16, 16)] = jnp.zeros((16,), jnp.float32)
        return carry

    lax.fori_loop(0, CH, zrow, 0)
    for k in range(NROWS_TILE // FLUSH_CHUNK):
        off = s * NROWS_TILE + k * FLUSH_CHUNK
        pltpu.sync_copy(rows.at[0], agg_sh.at[pl.ds(off, FLUSH_CHUNK)])
    plsc.subcore_barrier()

    n = TB + (s < TR).astype(jnp.int32)
    loc = c * CSCP + s * TB + jnp.minimum(s, TR)
    base = part * CPP + loc

    pltpu.sync_copy(
        idx_hbm.at[pl.ds(base * CH, TB * CH)], idx1.at[pl.ds(0, TB * CH)]
    )

    @pl.when(n > TB)
    def _():
        pltpu.sync_copy(
            idx_hbm.at[pl.ds((base + TB) * CH, CH)],
            idx1.at[pl.ds(TB * CH, CH)],
        )

    def m_desc(slot, j):
        return pltpu.make_async_copy(
            msg_hbm.at[pl.ds((loc + j) * CH, CH)], rows.at[slot],
            sfA if slot == 0 else sfB,
        )

    def a_desc(slot, j):
        return pltpu.make_async_copy(
            rows.at[slot], agg_sh.at[idx1.at[pl.ds(j * CH, CH)]],
            saA if slot == 0 else saB,
        )

    def fire_f(slot, j):
        @pl.when(j < n)
        def _():
            m_desc(slot, j).start()

    def drain_f(slot, j):
        @pl.when(j < n)
        def _():
            m_desc(slot, j).wait()

    def fire_a(slot, j):
        @pl.when(j < n)
        def _():
            a_desc(slot, j).start(add=True)

    def drain_a(slot, j):
        @pl.when((j >= 0) & (j < n))
        def _():
            a_desc(slot, j).wait()

    def group(gi, carry):
        q = gi * NSLOT_S
        for b in range(NSLOT_S):
            drain_a(b, q - NSLOT_S + b)
            fire_f(b, q + b)
        for b in range(NSLOT_S):
            drain_f(b, q + b)
            fire_a(b, q + b)
        return carry

    lax.fori_loop(0, NGROUP_S, group, 0)
    q = NGROUP_S * NSLOT_S
    for b in range(NSLOT_S):
        drain_a(b, q - NSLOT_S + b)
    plsc.subcore_barrier()

    pltpu.sync_copy(
        agg_sh.at[pl.ds(s * NROWS_TILE, NROWS_TILE)],
        out_hbm.at[c, pl.ds(s * NROWS_TILE, NROWS_TILE)],
    )


@functools.cache
def _scatter_kernel(part):
    return pl.kernel(
        functools.partial(_scatter_body, part),
        mesh=plsc.VectorSubcoreMesh(**_MESH),
        out_type=jax.ShapeDtypeStruct((NC, NPAD, D), jnp.float32),
        scratch_types=[
            pltpu.VMEM(((TB + 2) * CH,), jnp.int32),
            pltpu.VMEM((NSLOT_S, CH, D), jnp.float32),
            pltpu.VMEM_SHARED((NPAD, D), jnp.float32),
            pltpu.SemaphoreType.DMA,
            pltpu.SemaphoreType.DMA,
            pltpu.SemaphoreType.DMA,
            pltpu.SemaphoreType.DMA,
        ],
        name=f"gine_scatter_p{part}",
    )


def _final_body(x_ref, dx_ref, agg0_ref, agg1_ref, batch_ref, eps_ref,
                gnw_ref, gnb_ref, gns_ref, o_ref):
    agg = (agg0_ref[0, :N, :] + agg0_ref[1, :N, :]
           + agg1_ref[0, :N, :] + agg1_ref[1, :N, :])
    d = dx_ref[...]
    t = (1.0 + eps_ref[0, 0]) * d + agg
    t = t * (1.0 / (1.0 + jnp.exp(-t)))
    h = x_ref[...] + t

    b = batch_ref[...]
    oh = (b == lax.broadcasted_iota(jnp.int32, (1, G), 1)).astype(jnp.float32)
    cnt = jnp.maximum(jnp.sum(oh, axis=0, keepdims=True), 1.0)
    sums = lax.dot_general(
        oh, h, (((0,), (0,)), ((), ())), preferred_element_type=jnp.float32
    )
    mean = sums / cnt.T
    mrow = jnp.dot(oh, mean, preferred_element_type=jnp.float32)
    centered = h - mrow * gns_ref[...]
    var = (
        lax.dot_general(
            oh, centered * centered, (((0,), (0,)), ((), ())),
            preferred_element_type=jnp.float32,
        )
        / cnt.T
    )
    vrow = jnp.dot(oh, var, preferred_element_type=jnp.float32)
    o_ref[...] = gnw_ref[...] * centered * lax.rsqrt(vrow + 1e-5) + gnb_ref[...]


def _final_call(x, dx, agg_a, agg_b, batch2, eps2, gnw, gnb, gns):
    return pl.pallas_call(
        _final_body,
        in_specs=[
            pl.BlockSpec(memory_space=pltpu.VMEM),
            pl.BlockSpec(memory_space=pltpu.VMEM),
            pl.BlockSpec(memory_space=pltpu.VMEM),
            pl.BlockSpec(memory_space=pltpu.VMEM),
            pl.BlockSpec(memory_space=pltpu.VMEM),
            pl.BlockSpec(memory_space=pltpu.SMEM),
            pl.BlockSpec(memory_space=pltpu.VMEM),
            pl.BlockSpec(memory_space=pltpu.VMEM),
            pl.BlockSpec(memory_space=pltpu.VMEM),
        ],
        out_shape=jax.ShapeDtypeStruct((N, D), jnp.float32),
    )(x, dx, agg_a, agg_b, batch2, eps2, gnw, gnb, gns)


def kernel(x, e, batch, edge_index, lin1_w, lin1_b, conv_lin_w, conv_lin_b,
           eps, gn_weight, gn_bias, gn_mean_scale):
    src = edge_index[0].astype(jnp.int32)
    dst = edge_index[1].astype(jnp.int32)
    dx = _dx_call(x, lin1_w.T, lin1_b.reshape(1, D))
    cwt = conv_lin_w.T
    cb = conv_lin_b.reshape(1, D)
    gathered = [_gather_kernel(p)(dx, src) for p in range(NPART)]
    msgs = [_msg_call(p, gathered[p], e, cwt, cb) for p in range(NPART)]
    aggs = [_scatter_kernel(p)(msgs[p], dst) for p in range(NPART)]
    out = _final_call(
        x, dx, aggs[0], aggs[1],
        batch.astype(jnp.int32).reshape(N, 1),
        eps.reshape(1, 1),
        gn_weight.reshape(1, D),
        gn_bias.reshape(1, D),
        gn_mean_scale.reshape(1, D),
    )
    return out


# speedup vs baseline: 1.0210x; 1.0210x over previous
import functools

import jax
import jax.numpy as jnp
from jax import lax
from jax.experimental import pallas as pl
from jax.experimental.pallas import tpu as pltpu
from jax.experimental.pallas import tpu_sc as plsc

N = 10000
E = 320000
D = 128
G = 8
NC = 2
NS = 16
NW = NC * NS

CH = 128
NCHUNK = E // CH
NPART = 2
CPP = NCHUNK // NPART
EPP = CPP * CH
WB = CPP // NW
WR = CPP % NW
CSCP = CPP // NC
TB = CSCP // NS
TR = CSCP % NS
NPAD = 10240
NROWS_TILE = NPAD // NS
FLUSH_CHUNK = 128

NSLOT = 2
HALF = 1
NGROUP = -(-(WB + 1) // NSLOT)
NSLOT_S = 2
NGROUP_S = -(-(TB + 1) // NSLOT_S)

_MESH = dict(core_axis_name="c", subcore_axis_name="s",
             num_cores=NC, num_subcores=NS)


def _dx_body(x_ref, w_ref, b_ref, o_ref):
    o_ref[...] = (
        jnp.dot(x_ref[...], w_ref[...], preferred_element_type=jnp.float32)
        + b_ref[...]
    )


def _dx_call(x, w1t, b1):
    return pl.pallas_call(
        _dx_body,
        out_shape=jax.ShapeDtypeStruct((N, D), jnp.float32),
    )(x, w1t, b1)


def _gather_body(part, dx_hbm, idx_hbm, out_hbm, idx1, rows, dxs,
                 sgA, sgB, ssA, ssB):
    c = lax.axis_index("c")
    s = lax.axis_index("s")
    wid = s * NC + c
    n = WB + (wid < WR).astype(jnp.int32)
    loc = wid * WB + jnp.minimum(wid, WR)
    base = part * CPP + loc

    @pl.when(s < 10)
    def _():
        pltpu.sync_copy(
            dx_hbm.at[pl.ds(s * (N // 10), N // 10)],
            dxs.at[pl.ds(s * (N // 10), N // 10)],
        )

    pltpu.sync_copy(
        idx_hbm.at[pl.ds(base * CH, WB * CH)], idx1.at[pl.ds(0, WB * CH)]
    )
    plsc.subcore_barrier()

    @pl.when(n > WB)
    def _():
        pltpu.sync_copy(
            idx_hbm.at[pl.ds((base + WB) * CH, CH)],
            idx1.at[pl.ds(WB * CH, CH)],
        )

    def g_desc(slot, j):
        return pltpu.make_async_copy(
            dxs.at[idx1.at[pl.ds(j * CH, CH)]], rows.at[slot],
            sgA if slot < HALF else sgB,
        )

    def s_desc(slot, j):
        return pltpu.make_async_copy(
            rows.at[slot], out_hbm.at[pl.ds((loc + j) * CH, CH)],
            ssA if slot < HALF else ssB,
        )

    def fire_g(slot, j):
        @pl.when(j < n)
        def _():
            g_desc(slot, j).start()

    def drain_g(slot, j):
        @pl.when(j < n)
        def _():
            g_desc(slot, j).wait()

    def fire_s(slot, j):
        @pl.when(j < n)
        def _():
            s_desc(slot, j).start()

    def drain_s(slot, j):
        @pl.when((j >= 0) & (j < n))
        def _():
            s_desc(slot, j).wait()

    def group(gi, carry):
        q = gi * NSLOT
        for b in range(HALF):
            drain_s(b, q - NSLOT + b)
            fire_g(b, q + b)
        for b in range(HALF, NSLOT):
            drain_s(b, q - NSLOT + b)
            fire_g(b, q + b)
        for b in range(HALF):
            drain_g(b, q + b)
            fire_s(b, q + b)
        for b in range(HALF, NSLOT):
            drain_g(b, q + b)
            fire_s(b, q + b)
        return carry

    lax.fori_loop(0, NGROUP, group, 0)
    q = NGROUP * NSLOT
    for b in range(NSLOT):
        drain_s(b, q - NSLOT + b)


@functools.cache
def _gather_kernel(part):
    return pl.kernel(
        functools.partial(_gather_body, part),
        mesh=plsc.VectorSubcoreMesh(**_MESH),
        out_type=jax.ShapeDtypeStruct((EPP, D), jnp.float32),
        scratch_types=[
            pltpu.VMEM(((WB + 2) * CH,), jnp.int32),
            pltpu.VMEM((NSLOT, CH, D), jnp.float32),
            pltpu.VMEM_SHARED((N, D), jnp.float32),
            pltpu.SemaphoreType.DMA,
            pltpu.SemaphoreType.DMA,
            pltpu.SemaphoreType.DMA,
            pltpu.SemaphoreType.DMA,
        ],
        name=f"gine_gather_p{part}",
    )


def _msg_body(g_ref, e_ref, w_ref, b_ref, o_ref):
    v = (
        g_ref[...]
        + jnp.dot(e_ref[...], w_ref[...], preferred_element_type=jnp.float32)
        + b_ref[...]
    )
    o_ref[...] = v * (1.0 / (1.0 + jnp.exp(-v)))


def _msg_call(part, gathered, e, cwt, cb):
    blk = 8000
    grid = EPP // blk
    off = part * grid
    return pl.pallas_call(
        _msg_body,
        grid=(grid,),
        in_specs=[
            pl.BlockSpec((blk, D), lambda i: (i, 0)),
            pl.BlockSpec((blk, D), lambda i, off=off: (i + off, 0)),
            pl.BlockSpec((D, D), lambda i: (0, 0)),
            pl.BlockSpec((1, D), lambda i: (0, 0)),
        ],
        out_specs=pl.BlockSpec((blk, D), lambda i: (i, 0)),
        out_shape=jax.ShapeDtypeStruct((EPP, D), jnp.float32),
    )(gathered, e, cwt, cb)


def _scatter_body(part, msg_hbm, idx_hbm, out_hbm, idxb, rows, agg_sh,
                  sfA, sfB, saA, saB):
    c = lax.axis_index("c")
    s = lax.axis_index("s")

    def zrow(i, carry):
        for k in range(D // 16):
            rows[0, i, pl.ds(k * 16, 16)] = jnp.zeros((16,), jnp.float32)
        return carry

    lax.fori_loop(0, CH, zrow, 0)
    for k in range(NROWS_TILE // FLUSH_CHUNK):
        off = s * NROWS_TILE + k * FLUSH_CHUNK
        pltpu.sync_copy(rows.at[0], agg_sh.at[pl.ds(off, FLUSH_CHUNK)])
    plsc.subcore_barrier()

    n = TB + (s < TR).astype(jnp.int32)
    loc = c * CSCP + s * TB + jnp.minimum(s, TR)
    base = part * CPP + loc

    def i_desc(slot, j):
        return pltpu.make_async_copy(
            idx_hbm.at[pl.ds((base + j) * CH, CH)], idxb.at[slot],
            sfA if slot == 0 else sfB,
        )

    def m_desc(slot, j):
        return pltpu.make_async_copy(
            msg_hbm.at[pl.ds((loc + j) * CH, CH)], rows.at[slot],
            sfA if slot == 0 else sfB,
        )

    def a_desc(slot):
        return pltpu.make_async_copy(
            rows.at[slot], agg_sh.at[idxb.at[slot]],
            saA if slot == 0 else saB,
        )

    def fire_f(slot, j):
        @pl.when(j < n)
        def _():
            i_desc(slot, j).start()
            m_desc(slot, j).start()

    def drain_f(slot, j):
        @pl.when(j < n)
        def _():
            i_desc(slot, j).wait()
            m_desc(slot, j).wait()

    def fire_a(slot, j):
        @pl.when(j < n)
        def _():
            a_desc(slot).start(add=True)

    def drain_a(slot, j):
        @pl.when((j >= 0) & (j < n))
        def _():
            a_desc(slot).wait()

    def group(gi, carry):
        q = gi * NSLOT_S
        for b in range(NSLOT_S):
            drain_a(b, q - NSLOT_S + b)
            fire_f(b, q + b)
        for b in range(NSLOT_S):
            drain_f(b, q + b)
            fire_a(b, q + b)
        return carry

    lax.fori_loop(0, NGROUP_S, group, 0)
    q = NGROUP_S * NSLOT_S
    for b in range(NSLOT_S):
        drain_a(b, q - NSLOT_S + b)
    plsc.subcore_barrier()

    pltpu.sync_copy(
        agg_sh.at[pl.ds(s * NROWS_TILE, NROWS_TILE)],
        out_hbm.at[c, pl.ds(s * NROWS_TILE, NROWS_TILE)],
    )


@functools.cache
def _scatter_kernel(part):
    return pl.kernel(
        functools.partial(_scatter_body, part),
        mesh=plsc.VectorSubcoreMesh(**_MESH),
        out_type=jax.ShapeDtypeStruct((NC, NPAD, D), jnp.float32),
        scratch_types=[
            pltpu.VMEM((NSLOT_S, CH), jnp.int32),
            pltpu.VMEM((NSLOT_S, CH, D), jnp.float32),
            pltpu.VMEM_SHARED((NPAD, D), jnp.float32),
            pltpu.SemaphoreType.DMA,
            pltpu.SemaphoreType.DMA,
            pltpu.SemaphoreType.DMA,
            pltpu.SemaphoreType.DMA,
        ],
        name=f"gine_scatter_p{part}",
    )


def _final_body(x_ref, dx_ref, agg0_ref, agg1_ref, batch_ref, eps_ref,
                gnw_ref, gnb_ref, gns_ref, o_ref):
    agg = (agg0_ref[0, :N, :] + agg0_ref[1, :N, :]
           + agg1_ref[0, :N, :] + agg1_ref[1, :N, :])
    d = dx_ref[...]
    t = (1.0 + eps_ref[0, 0]) * d + agg
    t = t * (1.0 / (1.0 + jnp.exp(-t)))
    h = x_ref[...] + t

    b = batch_ref[...]
    oh = (b == lax.broadcasted_iota(jnp.int32, (1, G), 1)).astype(jnp.float32)
    cnt = jnp.maximum(jnp.sum(oh, axis=0, keepdims=True), 1.0)
    sums = lax.dot_general(
        oh, h, (((0,), (0,)), ((), ())), preferred_element_type=jnp.float32
    )
    mean = sums / cnt.T
    mrow = jnp.dot(oh, mean, preferred_element_type=jnp.float32)
    centered = h - mrow * gns_ref[...]
    var = (
        lax.dot_general(
            oh, centered * centered, (((0,), (0,)), ((), ())),
            preferred_element_type=jnp.float32,
        )
        / cnt.T
    )
    vrow = jnp.dot(oh, var, preferred_element_type=jnp.float32)
    o_ref[...] = gnw_ref[...] * centered * lax.rsqrt(vrow + 1e-5) + gnb_ref[...]


def _final_call(x, dx, agg_a, agg_b, batch2, eps2, gnw, gnb, gns):
    return pl.pallas_call(
        _final_body,
        in_specs=[
            pl.BlockSpec(memory_space=pltpu.VMEM),
            pl.BlockSpec(memory_space=pltpu.VMEM),
            pl.BlockSpec(memory_space=pltpu.VMEM),
            pl.BlockSpec(memory_space=pltpu.VMEM),
            pl.BlockSpec(memory_space=pltpu.VMEM),
            pl.BlockSpec(memory_space=pltpu.SMEM),
            pl.BlockSpec(memory_space=pltpu.VMEM),
            pl.BlockSpec(memory_space=pltpu.VMEM),
            pl.BlockSpec(memory_space=pltpu.VMEM),
        ],
        out_shape=jax.ShapeDtypeStruct((N, D), jnp.float32),
    )(x, dx, agg_a, agg_b, batch2, eps2, gnw, gnb, gns)


def kernel(x, e, batch, edge_index, lin1_w, lin1_b, conv_lin_w, conv_lin_b,
           eps, gn_weight, gn_bias, gn_mean_scale):
    src = edge_index[0].astype(jnp.int32)
    dst = edge_index[1].astype(jnp.int32)
    dx = _dx_call(x, lin1_w.T, lin1_b.reshape(1, D))
    cwt = conv_lin_w.T
    cb = conv_lin_b.reshape(1, D)
    gathered = [_gather_kernel(p)(dx, src) for p in range(NPART)]
    msgs = [_msg_call(p, gathered[p], e, cwt, cb) for p in range(NPART)]
    aggs = [_scatter_kernel(p)(msgs[p], dst) for p in range(NPART)]
    out = _final_call(
        x, dx, aggs[0], aggs[1],
        batch.astype(jnp.int32).reshape(N, 1),
        eps.reshape(1, 1),
        gn_weight.reshape(1, D),
        gn_bias.reshape(1, D),
        gn_mean_scale.reshape(1, D),
    )
    return out


# speedup vs baseline: 1.0266x; 1.0055x over previous
import functools

import jax
import jax.numpy as jnp
from jax import lax
from jax.experimental import pallas as pl
from jax.experimental.pallas import tpu as pltpu
from jax.experimental.pallas import tpu_sc as plsc

N = 10000
E = 320000
D = 128
G = 8
NC = 2
NS = 16
NW = NC * NS

CH = 128
NCHUNK = E // CH
NPART = 2
CPP = NCHUNK // NPART
EPP = CPP * CH
WB = CPP // NW
WR = CPP % NW
CSCP = CPP // NC
TB = CSCP // NS
TR = CSCP % NS
NPAD = 10240
NROWS_TILE = NPAD // NS
FLUSH_CHUNK = 128

NSLOT = 2
HALF = 1
NGROUP = -(-(WB + 1) // NSLOT)
NSLOT_S = 2
NGROUP_S = -(-(TB + 1) // NSLOT_S)

_MESH = dict(core_axis_name="c", subcore_axis_name="s",
             num_cores=NC, num_subcores=NS)


def _dx_body(x_ref, w_ref, b_ref, o_ref):
    o_ref[...] = (
        jnp.dot(x_ref[...], w_ref[...], preferred_element_type=jnp.float32)
        + b_ref[...]
    )


def _dx_call(x, w1t, b1):
    return pl.pallas_call(
        _dx_body,
        out_shape=jax.ShapeDtypeStruct((N, D), jnp.float32),
    )(x, w1t, b1)


def _gather_body(part, dx_hbm, idx_hbm, out_hbm, idx1, rows, dxs,
                 sgA, sgB, ssA, ssB):
    c = lax.axis_index("c")
    s = lax.axis_index("s")
    wid = s * NC + c
    n = WB + (wid < WR).astype(jnp.int32)
    loc = wid * WB + jnp.minimum(wid, WR)
    base = part * CPP + loc

    @pl.when(s < 10)
    def _():
        pltpu.sync_copy(
            dx_hbm.at[pl.ds(s * (N // 10), N // 10)],
            dxs.at[pl.ds(s * (N // 10), N // 10)],
        )

    pltpu.sync_copy(
        idx_hbm.at[pl.ds(base * CH, WB * CH)], idx1.at[pl.ds(0, WB * CH)]
    )
    plsc.subcore_barrier()

    @pl.when(n > WB)
    def _():
        pltpu.sync_copy(
            idx_hbm.at[pl.ds((base + WB) * CH, CH)],
            idx1.at[pl.ds(WB * CH, CH)],
        )

    def g_desc(slot, j):
        return pltpu.make_async_copy(
            dxs.at[idx1.at[pl.ds(j * CH, CH)]], rows.at[slot],
            sgA if slot < HALF else sgB,
        )

    def s_desc(slot, j):
        return pltpu.make_async_copy(
            rows.at[slot], out_hbm.at[pl.ds((loc + j) * CH, CH)],
            ssA if slot < HALF else ssB,
        )

    def fire_g(slot, j):
        @pl.when(j < n)
        def _():
            g_desc(slot, j).start()

    def drain_g(slot, j):
        @pl.when(j < n)
        def _():
            g_desc(slot, j).wait()

    def fire_s(slot, j):
        @pl.when(j < n)
        def _():
            s_desc(slot, j).start()

    def drain_s(slot, j):
        @pl.when((j >= 0) & (j < n))
        def _():
            s_desc(slot, j).wait()

    def group(gi, carry):
        q = gi * NSLOT
        for b in range(HALF):
            drain_s(b, q - NSLOT + b)
            fire_g(b, q + b)
        for b in range(HALF, NSLOT):
            drain_s(b, q - NSLOT + b)
            fire_g(b, q + b)
        for b in range(HALF):
            drain_g(b, q + b)
            fire_s(b, q + b)
        for b in range(HALF, NSLOT):
            drain_g(b, q + b)
            fire_s(b, q + b)
        return carry

    lax.fori_loop(0, NGROUP, group, 0)
    q = NGROUP * NSLOT
    for b in range(NSLOT):
        drain_s(b, q - NSLOT + b)


@functools.cache
def _gather_kernel(part):
    return pl.kernel(
        functools.partial(_gather_body, part),
        mesh=plsc.VectorSubcoreMesh(**_MESH),
        out_type=jax.ShapeDtypeStruct((EPP, D), jnp.float32),
        scratch_types=[
            pltpu.VMEM(((WB + 2) * CH,), jnp.int32),
            pltpu.VMEM((NSLOT, CH, D), jnp.float32),
            pltpu.VMEM_SHARED((N, D), jnp.float32),
            pltpu.SemaphoreType.DMA,
            pltpu.SemaphoreType.DMA,
            pltpu.SemaphoreType.DMA,
            pltpu.SemaphoreType.DMA,
        ],
        name=f"gine_gather_p{part}",
    )


def _msg_body(g_ref, e_ref, w_ref, b_ref, o_ref):
    v = (
        g_ref[...]
        + jnp.dot(e_ref[...], w_ref[...], preferred_element_type=jnp.float32)
        + b_ref[...]
    )
    o_ref[...] = v * (1.0 / (1.0 + jnp.exp(-v)))


def _msg_call(part, gathered, e, cwt, cb):
    blk = 16000
    grid = EPP // blk
    off = part * grid
    return pl.pallas_call(
        _msg_body,
        grid=(grid,),
        in_specs=[
            pl.BlockSpec((blk, D), lambda i: (i, 0)),
            pl.BlockSpec((blk, D), lambda i, off=off: (i + off, 0)),
            pl.BlockSpec((D, D), lambda i: (0, 0)),
            pl.BlockSpec((1, D), lambda i: (0, 0)),
        ],
        out_specs=pl.BlockSpec((blk, D), lambda i: (i, 0)),
        out_shape=jax.ShapeDtypeStruct((EPP, D), jnp.float32),
    )(gathered, e, cwt, cb)


def _scatter_body(part, msg_hbm, idx_hbm, out_hbm, idxb, rows, agg_sh,
                  sfA, sfB, saA, saB):
    c = lax.axis_index("c")
    s = lax.axis_index("s")

    def zrow(i, carry):
        for k in range(D // 16):
            rows[0, i, pl.ds(k * 16, 16)] = jnp.zeros((16,), jnp.float32)
        return carry

    lax.fori_loop(0, CH, zrow, 0)
    for k in range(NROWS_TILE // FLUSH_CHUNK):
        off = s * NROWS_TILE + k * FLUSH_CHUNK
        pltpu.sync_copy(rows.at[0], agg_sh.at[pl.ds(off, FLUSH_CHUNK)])
    plsc.subcore_barrier()

    n = TB + (s < TR).astype(jnp.int32)
    loc = c * CSCP + s * TB + jnp.minimum(s, TR)
    base = part * CPP + loc

    def i_desc(slot, j):
        return pltpu.make_async_copy(
            idx_hbm.at[pl.ds((base + j) * CH, CH)], idxb.at[slot],
            sfA if slot == 0 else sfB,
        )

    def m_desc(slot, j):
        return pltpu.make_async_copy(
            msg_hbm.at[pl.ds((loc + j) * CH, CH)], rows.at[slot],
            sfA if slot == 0 else sfB,
        )

    def a_desc(slot):
        return pltpu.make_async_copy(
            rows.at[slot], agg_sh.at[idxb.at[slot]],
            saA if slot == 0 else saB,
        )

    def fire_f(slot, j):
        @pl.when(j < n)
        def _():
            i_desc(slot, j).start()
            m_desc(slot, j).start()

    def drain_f(slot, j):
        @pl.when(j < n)
        def _():
            i_desc(slot, j).wait()
            m_desc(slot, j).wait()

    def fire_a(slot, j):
        @pl.when(j < n)
        def _():
            a_desc(slot).start(add=True)

    def drain_a(slot, j):
        @pl.when((j >= 0) & (j < n))
        def _():
            a_desc(slot).wait()

    def group(gi, carry):
        q = gi * NSLOT_S
        for b in range(NSLOT_S):
            drain_a(b, q - NSLOT_S + b)
            fire_f(b, q + b)
        for b in range(NSLOT_S):
            drain_f(b, q + b)
            fire_a(b, q + b)
        return carry

    lax.fori_loop(0, NGROUP_S, group, 0)
    q = NGROUP_S * NSLOT_S
    for b in range(NSLOT_S):
        drain_a(b, q - NSLOT_S + b)
    plsc.subcore_barrier()

    pltpu.sync_copy(
        agg_sh.at[pl.ds(s * NROWS_TILE, NROWS_TILE)],
        out_hbm.at[c, pl.ds(s * NROWS_TILE, NROWS_TILE)],
    )


@functools.cache
def _scatter_kernel(part):
    return pl.kernel(
        functools.partial(_scatter_body, part),
        mesh=plsc.VectorSubcoreMesh(**_MESH),
        out_type=jax.ShapeDtypeStruct((NC, NPAD, D), jnp.float32),
        scratch_types=[
            pltpu.VMEM((NSLOT_S, CH), jnp.int32),
            pltpu.VMEM((NSLOT_S, CH, D), jnp.float32),
            pltpu.VMEM_SHARED((NPAD, D), jnp.float32),
            pltpu.SemaphoreType.DMA,
            pltpu.SemaphoreType.DMA,
            pltpu.SemaphoreType.DMA,
            pltpu.SemaphoreType.DMA,
        ],
        name=f"gine_scatter_p{part}",
    )


def _final_body(x_ref, dx_ref, agg0_ref, agg1_ref, batch_ref, eps_ref,
                gnw_ref, gnb_ref, gns_ref, o_ref):
    agg = (agg0_ref[0, :N, :] + agg0_ref[1, :N, :]
           + agg1_ref[0, :N, :] + agg1_ref[1, :N, :])
    d = dx_ref[...]
    t = (1.0 + eps_ref[0, 0]) * d + agg
    t = t * (1.0 / (1.0 + jnp.exp(-t)))
    h = x_ref[...] + t

    b = batch_ref[...]
    oh = (b == lax.broadcasted_iota(jnp.int32, (1, G), 1)).astype(jnp.float32)
    cnt = jnp.maximum(jnp.sum(oh, axis=0, keepdims=True), 1.0)
    sums = lax.dot_general(
        oh, h, (((0,), (0,)), ((), ())), preferred_element_type=jnp.float32
    )
    mean = sums / cnt.T
    mrow = jnp.dot(oh, mean, preferred_element_type=jnp.float32)
    centered = h - mrow * gns_ref[...]
    var = (
        lax.dot_general(
            oh, centered * centered, (((0,), (0,)), ((), ())),
            preferred_element_type=jnp.float32,
        )
        / cnt.T
    )
    vrow = jnp.dot(oh, var, preferred_element_type=jnp.float32)
    o_ref[...] = gnw_ref[...] * centered * lax.rsqrt(vrow + 1e-5) + gnb_ref[...]


def _final_call(x, dx, agg_a, agg_b, batch2, eps2, gnw, gnb, gns):
    return pl.pallas_call(
        _final_body,
        in_specs=[
            pl.BlockSpec(memory_space=pltpu.VMEM),
            pl.BlockSpec(memory_space=pltpu.VMEM),
            pl.BlockSpec(memory_space=pltpu.VMEM),
            pl.BlockSpec(memory_space=pltpu.VMEM),
            pl.BlockSpec(memory_space=pltpu.VMEM),
            pl.BlockSpec(memory_space=pltpu.SMEM),
            pl.BlockSpec(memory_space=pltpu.VMEM),
            pl.BlockSpec(memory_space=pltpu.VMEM),
            pl.BlockSpec(memory_space=pltpu.VMEM),
        ],
        out_shape=jax.ShapeDtypeStruct((N, D), jnp.float32),
    )(x, dx, agg_a, agg_b, batch2, eps2, gnw, gnb, gns)


def kernel(x, e, batch, edge_index, lin1_w, lin1_b, conv_lin_w, conv_lin_b,
           eps, gn_weight, gn_bias, gn_mean_scale):
    src = edge_index[0].astype(jnp.int32)
    dst = edge_index[1].astype(jnp.int32)
    dx = _dx_call(x, lin1_w.T, lin1_b.reshape(1, D))
    cwt = conv_lin_w.T
    cb = conv_lin_b.reshape(1, D)
    gathered = [_gather_kernel(p)(dx, src) for p in range(NPART)]
    msgs = [_msg_call(p, gathered[p], e, cwt, cb) for p in range(NPART)]
    aggs = [_scatter_kernel(p)(msgs[p], dst) for p in range(NPART)]
    out = _final_call(
        x, dx, aggs[0], aggs[1],
        batch.astype(jnp.int32).reshape(N, 1),
        eps.reshape(1, 1),
        gn_weight.reshape(1, D),
        gn_bias.reshape(1, D),
        gn_mean_scale.reshape(1, D),
    )
    return out
